# baseline (speedup 1.0000x reference)
import functools

import jax
import jax.numpy as jnp
from jax import lax
from jax.experimental import pallas as pl
from jax.experimental.pallas import tpu as pltpu
from jax.experimental.pallas import tpu_sc as plsc


def _make_sc_embed(B, T, V, D):
    info = plsc.get_sparse_core_info()
    NC, NS, L = info.num_cores, info.num_subcores, info.num_lanes
    NW = NC * NS

    assert T % NW == 0, (T, NW)
    P = T // NW
    C = 8
    HP = 16
    NBUF = 3
    assert P % C == 0 and D % L == 0 and P % HP == 0 and HP % C == 0
    NG = P // C
    assert NG >= NBUF

    mesh = plsc.VectorSubcoreMesh(core_axis_name="c", subcore_axis_name="s")

    tok_scratch = [pltpu.VMEM((C, D), jnp.float32) for _ in range(NBUF * B)]

    @functools.partial(
        pl.kernel,
        mesh=mesh,
        out_type=jax.ShapeDtypeStruct((B * T, D), jnp.float32),
        scratch_types=[
            pltpu.VMEM((B * P,), jnp.int32),
            pltpu.VMEM((HP, D), jnp.float32),
            *tok_scratch,
            *[pltpu.SemaphoreType.DMA for _ in range(2 * NBUF)],
        ],
    )
    def k(idx_hbm, tok_hbm, pos_hbm, out_hbm, idx_v, pos_h, *rest):
        bufs = [list(rest[i * B:(i + 1) * B]) for i in range(NBUF)]
        gsems = rest[NBUF * B:NBUF * B + NBUF]
        osems = rest[NBUF * B + NBUF:NBUF * B + 2 * NBUF]

        wid = lax.axis_index("s") * NC + lax.axis_index("c")
        pbase = wid * P

        for b in range(B):
            pltpu.sync_copy(idx_hbm.at[pl.ds(b * T + pbase, P)],
                            idx_v.at[pl.ds(b * P, P)])

        gdesc = [None] * NG
        odesc = [None] * NG

        def start_group(g):
            p = g % NBUF
            ds = []
            for b in range(B):
                d = pltpu.make_async_copy(
                    tok_hbm.at[idx_v.at[pl.ds(b * P + g * C, C)]],
                    bufs[p][b], gsems[p])
                d.start()
                ds.append(d)
            gdesc[g] = ds

        def start_out(g):
            p = g % NBUF
            ds = []
            for b in range(B):
                d = pltpu.make_async_copy(
                    bufs[p][b],
                    out_hbm.at[pl.ds(b * T + pbase + g * C, C)],
                    osems[p])
                d.start()
                ds.append(d)
            odesc[g] = ds

        cur_half = -1
        for g0 in range(NBUF - 1):
            start_group(g0)
        for g in range(NG):
            p = g % NBUF
            if g + NBUF - 1 < NG:
                if g >= 1:
                    for d in odesc[g - 1]:
                        d.wait()
                start_group(g + NBUF - 1)

            h = (g * C) // HP
            if h != cur_half:
                pltpu.sync_copy(pos_hbm.at[pl.ds(pbase + h * HP, HP)],
                                pos_h)
                cur_half = h
            off_h = g * C - h * HP

            for d in gdesc[g]:
                d.wait()

            bufs4 = bufs[p]

            def row_body(i, _, bufs4=bufs4, off_h=off_h):
                for j in range(D // L):
                    sl = pl.ds(j * L, L)
                    pv = pos_h[off_h + i, sl]
                    for b in range(B):
                        plsc.addupdate(bufs4[b].at[i, sl], pv)
                return 0

            lax.fori_loop(0, C, row_body, 0)
            start_out(g)

        for g in range(max(0, NG - NBUF), NG):
            for d in odesc[g]:
                d.wait()

    return k


def kernel(idx, token_embedding, position_embedding):
    B, T = idx.shape
    V, D = token_embedding.shape
    k = _make_sc_embed(B, T, V, D)
    out = k(idx.reshape(B * T).astype(jnp.int32),
            token_embedding,
            position_embedding[:T])
    return out.reshape(B, T, D)


# speedup vs baseline: 1.0523x; 1.0523x over previous
import functools

import jax
import jax.numpy as jnp
from jax import lax
from jax.experimental import pallas as pl
from jax.experimental.pallas import tpu as pltpu
from jax.experimental.pallas import tpu_sc as plsc


def _make_sc_embed(B, T, V, D):
    info = plsc.get_sparse_core_info()
    NC, NS, L = info.num_cores, info.num_subcores, info.num_lanes
    NW = NC * NS

    assert T % NW == 0, (T, NW)
    P = T // NW
    C = 8
    HP = 32
    NBUF = 2
    assert P % C == 0 and D % L == 0 and P % HP == 0 and HP % C == 0
    NG = P // C
    assert NG >= NBUF

    mesh = plsc.VectorSubcoreMesh(core_axis_name="c", subcore_axis_name="s")

    tok_scratch = [pltpu.VMEM((C, D), jnp.float32) for _ in range(NBUF * B)]

    @functools.partial(
        pl.kernel,
        mesh=mesh,
        out_type=jax.ShapeDtypeStruct((B * T, D), jnp.float32),
        scratch_types=[
            pltpu.VMEM((B * P,), jnp.int32),
            pltpu.VMEM((HP, D), jnp.float32),
            *tok_scratch,
            *[pltpu.SemaphoreType.DMA for _ in range(2 * NBUF)],
        ],
    )
    def k(idx_hbm, tok_hbm, pos_hbm, out_hbm, idx_v, pos_h, *rest):
        bufs = [list(rest[i * B:(i + 1) * B]) for i in range(NBUF)]
        gsems = rest[NBUF * B:NBUF * B + NBUF]
        osems = rest[NBUF * B + NBUF:NBUF * B + 2 * NBUF]

        wid = lax.axis_index("s") * NC + lax.axis_index("c")
        pbase = wid * P

        idx_ds = []
        for b in range(B):
            d = pltpu.make_async_copy(idx_hbm.at[pl.ds(b * T + pbase, P)],
                                      idx_v.at[pl.ds(b * P, P)], gsems[0])
            d.start()
            idx_ds.append(d)
        for d in idx_ds:
            d.wait()

        gdesc = [None] * NG
        odesc = [None] * NG

        def start_group(g):
            p = g % NBUF
            ds = []
            for b in range(B):
                d = pltpu.make_async_copy(
                    tok_hbm.at[idx_v.at[pl.ds(b * P + g * C, C)]],
                    bufs[p][b], gsems[p])
                d.start()
                ds.append(d)
            gdesc[g] = ds

        def start_out(g):
            p = g % NBUF
            ds = []
            for b in range(B):
                d = pltpu.make_async_copy(
                    bufs[p][b],
                    out_hbm.at[pl.ds(b * T + pbase + g * C, C)],
                    osems[p])
                d.start()
                ds.append(d)
            odesc[g] = ds

        cur_half = -1
        for g0 in range(NBUF - 1):
            start_group(g0)
        for g in range(NG):
            p = g % NBUF
            if g + NBUF - 1 < NG:
                if g >= 1:
                    for d in odesc[g - 1]:
                        d.wait()
                start_group(g + NBUF - 1)

            h = (g * C) // HP
            if h != cur_half:
                pltpu.sync_copy(pos_hbm.at[pl.ds(pbase + h * HP, HP)],
                                pos_h)
                cur_half = h
            off_h = g * C - h * HP

            for d in gdesc[g]:
                d.wait()

            bufs4 = bufs[p]

            def row_body(i, _, bufs4=bufs4, off_h=off_h):
                for j in range(D // L):
                    sl = pl.ds(j * L, L)
                    pv = pos_h[off_h + i, sl]
                    for b in range(B):
                        plsc.addupdate(bufs4[b].at[i, sl], pv)
                return 0

            lax.fori_loop(0, C, row_body, 0)
            start_out(g)

        for g in range(max(0, NG - NBUF), NG):
            for d in odesc[g]:
                d.wait()

    return k


def kernel(idx, token_embedding, position_embedding):
    B, T = idx.shape
    V, D = token_embedding.shape
    k = _make_sc_embed(B, T, V, D)
    out = k(idx.reshape(B * T).astype(jnp.int32),
            token_embedding,
            position_embedding[:T])
    return out.reshape(B, T, D)
